# Initial kernel scaffold; baseline (speedup 1.0000x reference)
import jax
import jax.numpy as jnp
from jax.experimental import pallas as pl


def kernel(x, edge_index, W1_l, b1, W1_r, W2_l, b2, W2_r):
    raise NotImplementedError("write your pallas kernel here")


# speedup vs baseline: 3.8366x; 3.8366x over previous
import functools

import jax
import jax.numpy as jnp
from jax import lax
from jax.experimental import pallas as pl
from jax.experimental.pallas import tpu as pltpu
from jax.experimental.pallas import tpu_sc as plsc

N_NODES = 10000
N_EDGES = 320000
D = 128

NC = 2
NS = 16
NW = NC * NS

B = 128
EPW = N_EDGES // NW
CH = 80
EPW_PAD = CH * B
CHB = 8
NP = 10112
RPT = NP // NS


@functools.cache
def _make_agg():
    mesh = plsc.VectorSubcoreMesh(core_axis_name="c", subcore_axis_name="s")

    @functools.partial(
        pl.kernel,
        out_type=(jax.ShapeDtypeStruct((NC, NP, D), jnp.float32),),
        mesh=mesh,
        scratch_types=[
            pltpu.VMEM((CHB, B), jnp.int32),
            pltpu.VMEM((CHB, B), jnp.int32),
            pltpu.VMEM((B, D), jnp.float32),
            pltpu.VMEM((B, D), jnp.float32),
            pltpu.VMEM_SHARED((NP, D), jnp.float32),
            pltpu.SemaphoreType.DMA,
            pltpu.SemaphoreType.DMA,
        ],
        name="sc_agg")
    def body(x_hbm, src_hbm, dst_hbm, zeros_hbm, agg_out,
             src_v, dst_v, buf0, buf1, acc, sem0, sem1):
        c = lax.axis_index("c")
        s = lax.axis_index("s")
        w = s * NC + c

        rows = pl.ds(s * RPT, RPT)
        pltpu.sync_copy(zeros_hbm.at[rows], acc.at[rows])
        plsc.subcore_barrier()

        def macro(m, _):
            pltpu.sync_copy(src_hbm.at[w, pl.ds(m * CHB, CHB)], src_v)
            pltpu.sync_copy(dst_hbm.at[w, pl.ds(m * CHB, CHB)], dst_v)

            def step(i, _):
                j0 = 2 * i
                j1 = j0 + 1
                cp0 = pltpu.async_copy(x_hbm.at[src_v.at[j0]], buf0, sem0)
                cp1 = pltpu.async_copy(x_hbm.at[src_v.at[j1]], buf1, sem1)
                cp0.wait()
                pltpu.sync_copy(buf0, acc.at[dst_v.at[j0]], add=True)
                cp1.wait()
                pltpu.sync_copy(buf1, acc.at[dst_v.at[j1]], add=True)
                return _

            lax.fori_loop(0, CHB // 2, step, None)
            return _

        lax.fori_loop(0, CH // CHB, macro, None)
        plsc.subcore_barrier()

        pltpu.sync_copy(acc.at[rows], agg_out.at[c, rows])

    return body


@functools.cache
def _make_count():
    mesh = plsc.VectorSubcoreMesh(core_axis_name="c", subcore_axis_name="s")

    @functools.partial(
        pl.kernel,
        out_type=(jax.ShapeDtypeStruct((NC, NP, 16), jnp.float32),),
        mesh=mesh,
        scratch_types=[
            pltpu.VMEM((CHB, B), jnp.int32),
            pltpu.VMEM((B, 16), jnp.float32),
            pltpu.VMEM_SHARED((NP, 16), jnp.float32),
        ],
        name="sc_count")
    def body(dst_hbm, zcnt_hbm, cnt_out, dst_v, ones_v, cacc):
        c = lax.axis_index("c")
        s = lax.axis_index("s")
        w = s * NC + c

        rows = pl.ds(s * RPT, RPT)
        pltpu.sync_copy(zcnt_hbm.at[rows], cacc.at[rows])
        ones16 = jnp.ones((16,), jnp.float32)
        for i in range(B):
            ones_v[i] = ones16
        plsc.subcore_barrier()

        def macro(m, _):
            pltpu.sync_copy(dst_hbm.at[w, pl.ds(m * CHB, CHB)], dst_v)

            def step(j, _):
                pltpu.sync_copy(ones_v, cacc.at[dst_v.at[j]], add=True)
                return _

            lax.fori_loop(0, CHB, step, None)
            return _

        lax.fori_loop(0, CH // CHB, macro, None)
        plsc.subcore_barrier()
        pltpu.sync_copy(cacc.at[rows], cnt_out.at[c, rows])

    return body


def _tc_inv_cnt_body(cntp_ref, out_ref):
    cnt = cntp_ref[0, :, :1] + cntp_ref[1, :, :1]
    out_ref[...] = 1.0 / jnp.maximum(cnt, 1.0)


def _tc_inv_cnt(cntp):
    return pl.pallas_call(
        _tc_inv_cnt_body,
        out_shape=jax.ShapeDtypeStruct((NP, 1), jnp.float32),
        name="tc_inv_cnt",
    )(cntp)


def _tc_layer_body(aggp_ref, inv_ref, x_ref, wl_ref, b_ref, wr_ref, out_ref,
                   *, last):
    agg = aggp_ref[0] + aggp_ref[1]
    mean = agg * inv_ref[...]
    h = (jnp.dot(mean, wl_ref[...], preferred_element_type=jnp.float32)
         + b_ref[...]
         + jnp.dot(x_ref[...], wr_ref[...], preferred_element_type=jnp.float32))
    h = jnp.maximum(h, 0.0)
    if last:
        m = jnp.max(h, axis=-1, keepdims=True)
        z = h - m
        h = z - jnp.log(jnp.sum(jnp.exp(z), axis=-1, keepdims=True))
    out_ref[...] = h


BN = 400


def _tc_layer(aggp, inv, x, wlT, b, wrT, *, last):
    grid = (N_NODES // BN,)
    return pl.pallas_call(
        functools.partial(_tc_layer_body, last=last),
        grid=grid,
        in_specs=[
            pl.BlockSpec((NC, BN, D), lambda i: (0, i, 0)),
            pl.BlockSpec((BN, 1), lambda i: (i, 0)),
            pl.BlockSpec((BN, D), lambda i: (i, 0)),
            pl.BlockSpec((D, D), lambda i: (0, 0)),
            pl.BlockSpec((1, D), lambda i: (0, 0)),
            pl.BlockSpec((D, D), lambda i: (0, 0)),
        ],
        out_specs=pl.BlockSpec((BN, D), lambda i: (i, 0)),
        out_shape=jax.ShapeDtypeStruct((N_NODES, D), jnp.float32),
        name=f"tc_sage_{'ls' if last else 'relu'}",
    )(aggp, inv, x, wlT, b, wrT)


def kernel(x, edge_index, W1_l, b1, W1_r, W2_l, b2, W2_r):
    src = edge_index[0].astype(jnp.int32).reshape(NW, EPW)
    dst = edge_index[1].astype(jnp.int32).reshape(NW, EPW)
    pad = EPW_PAD - EPW
    src3 = jnp.pad(src, ((0, 0), (0, pad))).reshape(NW, CH, B)
    dst3 = jnp.pad(dst, ((0, 0), (0, pad)),
                   constant_values=N_NODES).reshape(NW, CH, B)
    zeros = jnp.zeros((NP, D), jnp.float32)
    zcnt = jnp.zeros((NP, 16), jnp.float32)

    (cntp,) = _make_count()(dst3, zcnt)
    inv = _tc_inv_cnt(cntp)
    (aggp1,) = _make_agg()(x, src3, dst3, zeros)
    h = _tc_layer(aggp1, inv, x, W1_l.T, b1.reshape(1, D), W1_r.T, last=False)
    (aggp2,) = _make_agg()(h, src3, dst3, zeros)
    y = _tc_layer(aggp2, inv, h, W2_l.T, b2.reshape(1, D), W2_r.T, last=True)
    return jnp.transpose(y)[None]


# speedup vs baseline: 4.0831x; 1.0643x over previous
import functools

import jax
import jax.numpy as jnp
from jax import lax
from jax.experimental import pallas as pl
from jax.experimental.pallas import tpu as pltpu
from jax.experimental.pallas import tpu_sc as plsc

N_NODES = 10000
N_EDGES = 320000
D = 128

NC = 2
NS = 16
NW = NC * NS

B = 128
EPW = N_EDGES // NW
CH = 80
EPW_PAD = CH * B
CHB = 8
NP = 10112
RPT = NP // NS


@functools.cache
def _make_agg():
    mesh = plsc.VectorSubcoreMesh(core_axis_name="c", subcore_axis_name="s")

    @functools.partial(
        pl.kernel,
        out_type=(jax.ShapeDtypeStruct((NC, NP, D), jnp.float32),),
        mesh=mesh,
        scratch_types=[
            pltpu.VMEM((CHB, B), jnp.int32),
            pltpu.VMEM((CHB, B), jnp.int32),
            pltpu.VMEM((B, D), jnp.float32),
            pltpu.VMEM((B, D), jnp.float32),
            pltpu.VMEM_SHARED((NP, D), jnp.float32),
            pltpu.SemaphoreType.DMA,
            pltpu.SemaphoreType.DMA,
            pltpu.SemaphoreType.DMA,
            pltpu.SemaphoreType.DMA,
        ],
        name="sc_agg")
    def body(x_hbm, src_hbm, dst_hbm, zeros_hbm, agg_out,
             src_v, dst_v, buf0, buf1, acc, gs0, gs1, ss0, ss1):
        c = lax.axis_index("c")
        s = lax.axis_index("s")
        w = s * NC + c

        rows = pl.ds(s * RPT, RPT)
        pltpu.sync_copy(zeros_hbm.at[rows], acc.at[rows])
        plsc.subcore_barrier()

        bufs = (buf0, buf1)
        gsems = (gs0, gs1)
        ssems = (ss0, ss1)

        def macro(m, _):
            pltpu.sync_copy(src_hbm.at[w, pl.ds(m * CHB, CHB)], src_v)
            pltpu.sync_copy(dst_hbm.at[w, pl.ds(m * CHB, CHB)], dst_v)

            g = [pltpu.async_copy(x_hbm.at[src_v.at[0]], buf0, gs0),
                 pltpu.async_copy(x_hbm.at[src_v.at[1]], buf1, gs1)]
            tail = [None, None]
            for j in range(CHB):
                k = j % 2
                g[k].wait()
                s = pltpu.async_copy(bufs[k], acc.at[dst_v.at[j]], ssems[k],
                                     add=True)
                if j + 2 < CHB:
                    s.wait()
                    g[k] = pltpu.async_copy(x_hbm.at[src_v.at[j + 2]],
                                            bufs[k], gsems[k])
                else:
                    tail[k] = s
            tail[0].wait()
            tail[1].wait()
            return _

        lax.fori_loop(0, CH // CHB, macro, None)
        plsc.subcore_barrier()

        pltpu.sync_copy(acc.at[rows], agg_out.at[c, rows])

    return body


@functools.cache
def _make_count():
    mesh = plsc.VectorSubcoreMesh(core_axis_name="c", subcore_axis_name="s")

    @functools.partial(
        pl.kernel,
        out_type=(jax.ShapeDtypeStruct((NC, NP, 16), jnp.float32),),
        mesh=mesh,
        scratch_types=[
            pltpu.VMEM((CHB, B), jnp.int32),
            pltpu.VMEM((B, 16), jnp.float32),
            pltpu.VMEM_SHARED((NP, 16), jnp.float32),
        ],
        name="sc_count")
    def body(dst_hbm, zcnt_hbm, cnt_out, dst_v, ones_v, cacc):
        c = lax.axis_index("c")
        s = lax.axis_index("s")
        w = s * NC + c

        rows = pl.ds(s * RPT, RPT)
        pltpu.sync_copy(zcnt_hbm.at[rows], cacc.at[rows])
        ones16 = jnp.ones((16,), jnp.float32)
        for i in range(B):
            ones_v[i] = ones16
        plsc.subcore_barrier()

        def macro(m, _):
            pltpu.sync_copy(dst_hbm.at[w, pl.ds(m * CHB, CHB)], dst_v)

            def step(j, _):
                pltpu.sync_copy(ones_v, cacc.at[dst_v.at[j]], add=True)
                return _

            lax.fori_loop(0, CHB, step, None)
            return _

        lax.fori_loop(0, CH // CHB, macro, None)
        plsc.subcore_barrier()
        pltpu.sync_copy(cacc.at[rows], cnt_out.at[c, rows])

    return body


def _tc_inv_cnt_body(cntp_ref, out_ref):
    cnt = cntp_ref[0, :, :1] + cntp_ref[1, :, :1]
    out_ref[...] = 1.0 / jnp.maximum(cnt, 1.0)


def _tc_inv_cnt(cntp):
    return pl.pallas_call(
        _tc_inv_cnt_body,
        out_shape=jax.ShapeDtypeStruct((NP, 1), jnp.float32),
        name="tc_inv_cnt",
    )(cntp)


def _tc_layer_body(aggp_ref, inv_ref, x_ref, wl_ref, b_ref, wr_ref, out_ref,
                   *, last):
    agg = aggp_ref[0] + aggp_ref[1]
    mean = agg * inv_ref[...]
    h = (jnp.dot(mean, wl_ref[...], preferred_element_type=jnp.float32)
         + b_ref[...]
         + jnp.dot(x_ref[...], wr_ref[...], preferred_element_type=jnp.float32))
    h = jnp.maximum(h, 0.0)
    if last:
        m = jnp.max(h, axis=-1, keepdims=True)
        z = h - m
        h = z - jnp.log(jnp.sum(jnp.exp(z), axis=-1, keepdims=True))
    out_ref[...] = h


BN = 400


def _tc_layer(aggp, inv, x, wlT, b, wrT, *, last):
    grid = (N_NODES // BN,)
    return pl.pallas_call(
        functools.partial(_tc_layer_body, last=last),
        grid=grid,
        in_specs=[
            pl.BlockSpec((NC, BN, D), lambda i: (0, i, 0)),
            pl.BlockSpec((BN, 1), lambda i: (i, 0)),
            pl.BlockSpec((BN, D), lambda i: (i, 0)),
            pl.BlockSpec((D, D), lambda i: (0, 0)),
            pl.BlockSpec((1, D), lambda i: (0, 0)),
            pl.BlockSpec((D, D), lambda i: (0, 0)),
        ],
        out_specs=pl.BlockSpec((BN, D), lambda i: (i, 0)),
        out_shape=jax.ShapeDtypeStruct((N_NODES, D), jnp.float32),
        name=f"tc_sage_{'ls' if last else 'relu'}",
    )(aggp, inv, x, wlT, b, wrT)


def kernel(x, edge_index, W1_l, b1, W1_r, W2_l, b2, W2_r):
    src = edge_index[0].astype(jnp.int32).reshape(NW, EPW)
    dst = edge_index[1].astype(jnp.int32).reshape(NW, EPW)
    pad = EPW_PAD - EPW
    src3 = jnp.pad(src, ((0, 0), (0, pad))).reshape(NW, CH, B)
    dst3 = jnp.pad(dst, ((0, 0), (0, pad)),
                   constant_values=N_NODES).reshape(NW, CH, B)
    zeros = jnp.zeros((NP, D), jnp.float32)
    zcnt = jnp.zeros((NP, 16), jnp.float32)

    (cntp,) = _make_count()(dst3, zcnt)
    inv = _tc_inv_cnt(cntp)
    (aggp1,) = _make_agg()(x, src3, dst3, zeros)
    h = _tc_layer(aggp1, inv, x, W1_l.T, b1.reshape(1, D), W1_r.T, last=False)
    (aggp2,) = _make_agg()(h, src3, dst3, zeros)
    y = _tc_layer(aggp2, inv, h, W2_l.T, b2.reshape(1, D), W2_r.T, last=True)
    return jnp.transpose(y)[None]
